# Initial kernel scaffold; baseline (speedup 1.0000x reference)
import jax
import jax.numpy as jnp
from jax.experimental import pallas as pl


def kernel(coords, pairwise_dists, edge_ids, mask):
    raise NotImplementedError("write your pallas kernel here")


# speedup vs baseline: 233.0433x; 233.0433x over previous
import functools

import jax
import jax.numpy as jnp
from jax import lax
from jax.experimental import pallas as pl
from jax.experimental.pallas import tpu as pltpu, tpu_sc as plsc

NUM_RBF = 18
_MAGIC = 0x5F3759DF


def _rsqrt(x):
    i = plsc.bitcast(x, jnp.int32)
    y = plsc.bitcast(jnp.int32(_MAGIC) - (i >> 1), jnp.float32)
    y = y * (1.5 - 0.5 * x * y * y)
    y = y * (1.5 - 0.5 * x * y * y)
    return y


def _sqrt(x):
    return x * _rsqrt(jnp.maximum(x, 1e-35))


def _bf16r(x):
    u = plsc.bitcast(x, jnp.int32)
    r = (u + jnp.int32(0x7FFF) + ((u >> 16) & 1)) & jnp.int32(-65536)
    return plsc.bitcast(r, jnp.float32)


def _normalize3(v0, v1, v2):
    ss = v0 * v0 + v1 * v1 + v2 * v2
    inv = _rsqrt(jnp.maximum(ss, 1e-30))
    z = jnp.where(ss > 0.0, inv, 0.0)
    return v0 * z, v1 * z, v2 * z


def _cross(a, b):
    return (
        a[1] * b[2] - a[2] * b[1],
        a[2] * b[0] - a[0] * b[2],
        a[0] * b[1] - a[1] * b[0],
    )


def _sc_geo(ca_soa, dists, eidx, *, B, L, K):
    NT = 32
    TPB = NT // B
    LQ = L // TPB
    RPC = 8
    NCHUNK = LQ // RPC
    EPC = RPC * K
    QE = LQ * K
    CH = NUM_RBF + 7
    STW = EPC * CH

    mesh = plsc.VectorSubcoreMesh(core_axis_name="c", subcore_axis_name="s",
                                  num_cores=2, num_subcores=16)

    @functools.partial(
        pl.kernel,
        out_type=[
            jax.ShapeDtypeStruct((B * L * 3,), jnp.float32),
            jax.ShapeDtypeStruct((B * L * K * CH,), jnp.float32),
        ],
        mesh=mesh,
        compiler_params=pltpu.CompilerParams(needs_layout_passes=False),
        scratch_types=[
            pltpu.VMEM((12 * L,), jnp.float32),
            pltpu.VMEM((STW,), jnp.float32),
            pltpu.VMEM((STW,), jnp.float32),
            pltpu.VMEM((QE,), jnp.int32),
            pltpu.VMEM((QE,), jnp.float32),
            pltpu.SemaphoreType.DMA,
            pltpu.SemaphoreType.DMA,
            pltpu.SemaphoreType.DMA,
            pltpu.SemaphoreType.DMA,
        ],
    )
    def body(ca_hbm, dst_hbm, idx_hbm, node_hbm, edge_hbm,
             tab, st0, st1, idxq, dstq, si0, si1, so0, so1):
        cid = lax.axis_index("c")
        sid = lax.axis_index("s")
        wid = sid * 2 + cid
        b = wid // TPB
        q = wid % TPB

        qoff = b * (L * K) + q * QE
        in0 = pltpu.async_copy(idx_hbm.at[pl.ds(qoff, QE)], idxq, si0)
        in1 = pltpu.async_copy(dst_hbm.at[pl.ds(qoff, QE)], dstq, si1)

        pltpu.sync_copy(ca_hbm.at[pl.ds(b * 3 * L, 3 * L)], st1.at[pl.ds(0, 3 * L)])

        iota = lax.iota(jnp.int32, 16)
        eps = 1e-6

        @pl.loop(0, L // 16)
        def _node(g):
            lane = g * 16 + iota
            ms = [jnp.clip(lane + o, 0, L - 1) for o in (-1, 0, 1, 2)]
            xs = []
            for m in ms:
                xs.append([plsc.load_gather(st1, [jnp.int32(c * L) + m])
                           for c in range(3)])
            u2 = _normalize3(*[xs[1][c] - xs[0][c] for c in range(3)])
            u1 = _normalize3(*[xs[2][c] - xs[1][c] for c in range(3)])
            u0 = _normalize3(*[xs[3][c] - xs[2][c] for c in range(3)])
            n2 = _normalize3(*_cross(u2, u1))
            n1 = _normalize3(*_cross(u1, u0))
            cosA = -(u1[0] * u0[0] + u1[1] * u0[1] + u1[2] * u0[2])
            cosA = jnp.clip(cosA, -1 + eps, 1 - eps)
            cosD = n2[0] * n1[0] + n2[1] * n1[1] + n2[2] * n1[2]
            cosD = jnp.clip(cosD, -1 + eps, 1 - eps)
            sinA = _sqrt(1.0 - cosA * cosA)
            sgn = jnp.sign(u2[0] * n1[0] + u2[1] * n1[1] + u2[2] * n1[2])
            sinD = _sqrt(1.0 - cosD * cosD) * sgn
            o1 = _normalize3(u2[0] - u1[0], u2[1] - u1[1], u2[2] - u1[2])
            o3 = _cross(o1, n2)
            validf = jnp.where((lane >= 1) & (lane <= L - 3), 1.0, 0.0)
            orows = [o1[0], o1[1], o1[2], n2[0], n2[1], n2[2], o3[0], o3[1], o3[2]]
            for c in range(9):
                tab[pl.ds(c * L + g * 16, 16)] = orows[c] * validf
            for c in range(3):
                tab[pl.ds((9 + c) * L + g * 16, 16)] = xs[1][c]
            ad = [cosA, sinA * cosD, sinA * sinD]
            for c in range(3):
                plsc.store_scatter(st0, [lane * 3 + c], ad[c] * validf)

        pltpu.sync_copy(st0.at[pl.ds(q * (LQ * 3), LQ * 3)],
                        node_hbm.at[pl.ds(b * (L * 3) + q * (LQ * 3), LQ * 3)])

        in0.wait()
        in1.wait()

        mus = [m * (20.0 / (NUM_RBF - 1)) for m in range(NUM_RBF)]
        inv_sig = NUM_RBF / 20.0
        iota_ch = iota * CH
        stages = (st0, st1)
        sems = (so0, so1)

        def do_chunk(c, ph):
            st = stages[ph]
            sem = sems[ph]

            @pl.when(c >= 2)
            def _():
                pltpu.make_async_copy(st, edge_hbm.at[pl.ds(0, STW)], sem).wait()

            @pl.loop(0, RPC)
            def _row(j):
                labs = q * LQ + c * RPC + j
                own = [plsc.load_gather(tab, [jnp.full((16,), c2 * L, jnp.int32) + labs])
                       for c2 in range(12)]
                ownb = [_bf16r(own[c2]) for c2 in range(9)]
                ebase = c * EPC + j * K

                for g in range(K // 16):
                    off = ebase + g * 16
                    idxv = idxq[pl.ds(off, 16)]
                    Dv = dstq[pl.ds(off, 16)]
                    gj = [plsc.load_gather(tab, [jnp.int32(c2 * L) + idxv])
                          for c2 in range(12)]
                    outs = []
                    for m in range(NUM_RBF):
                        z = (Dv - mus[m]) * inv_sig
                        outs.append(jnp.exp(-(z * z)))
                    gjb = [_bf16r(gj[c2]) for c2 in range(9)]
                    d = [_bf16r(gj[9 + c2] - own[9 + c2]) for c2 in range(3)]
                    t = [ownb[r * 3 + 0] * d[0] + ownb[r * 3 + 1] * d[1]
                         + ownb[r * 3 + 2] * d[2] for r in range(3)]
                    outs.extend(_normalize3(*t))
                    R = [[ownb[0 * 3 + a] * gjb[0 * 3 + c2]
                          + ownb[1 * 3 + a] * gjb[1 * 3 + c2]
                          + ownb[2 * 3 + a] * gjb[2 * 3 + c2]
                          for c2 in range(3)] for a in range(3)]
                    tr0, tr1, tr2 = R[0][0], R[1][1], R[2][2]
                    a0 = jnp.abs(1.0 + tr0 - tr1 - tr2)
                    a1 = jnp.abs(1.0 - tr0 + tr1 - tr2)
                    a2 = jnp.abs(1.0 - tr0 - tr1 + tr2)
                    aw = jnp.maximum(1.0 + tr0 + tr1 + tr2, 0.0)
                    s0 = jnp.sign(R[2][1] - R[1][2])
                    s1 = jnp.sign(R[0][2] - R[2][0])
                    s2 = jnp.sign(R[1][0] - R[0][1])
                    qs = s0 * s0 * a0 + s1 * s1 * a1 + s2 * s2 * a2 + aw
                    invq = jnp.where(qs > 0.0, _rsqrt(jnp.maximum(qs, 1e-30)), 0.0)
                    outs.append(s0 * _sqrt(a0) * invq)
                    outs.append(s1 * _sqrt(a1) * invq)
                    outs.append(s2 * _sqrt(a2) * invq)
                    outs.append(_sqrt(aw) * invq)
                    sbase = (j * K + g * 16) * CH + iota_ch
                    for ch in range(CH):
                        plsc.store_scatter(st, [sbase + ch], outs[ch])

            row = b * (L // RPC) + q * NCHUNK + c
            pltpu.async_copy(st, edge_hbm.at[pl.ds(row * STW, STW)], sem)

        @pl.loop(0, NCHUNK // 2)
        def _chunks(c2):
            do_chunk(c2 * 2, 0)
            do_chunk(c2 * 2 + 1, 1)

        pltpu.make_async_copy(st0, edge_hbm.at[pl.ds(0, STW)], so0).wait()
        pltpu.make_async_copy(st1, edge_hbm.at[pl.ds(0, STW)], so1).wait()

    return body(ca_soa, dists, eidx)


def kernel(coords, pairwise_dists, edge_ids, mask):
    B, L, K = pairwise_dists.shape
    ca_soa = coords[:, :, 1, :].transpose(0, 2, 1).reshape(-1)
    dists = pairwise_dists.reshape(-1)
    eidx = edge_ids.astype(jnp.int32).reshape(-1)
    node_flat, edge_flat = _sc_geo(ca_soa, dists, eidx, B=B, L=L, K=K)
    return node_flat.reshape(B, L, 3), edge_flat.reshape(B, L, K, NUM_RBF + 7)
